# baseline (speedup 1.0000x reference)
import functools

import jax
import jax.numpy as jnp
from jax import lax
from jax.experimental import pallas as pl
from jax.experimental.pallas import tpu as pltpu
from jax.experimental.pallas import tpu_sc as plsc

VOCAB = 100000
EMB = 64
BATCH = 4096
FIELDS = 26

NC = 2
NS = 16
NW = NC * NS
E_PER_W = EMB // NW
GROUPS = BATCH // 16
XRING = 4

_mesh = plsc.VectorSubcoreMesh(core_axis_name="c", subcore_axis_name="s")


@functools.partial(
    pl.kernel,
    mesh=_mesh,
    out_type=jax.ShapeDtypeStruct((FIELDS, 8, 32, 8, 128), jnp.float32),
    compiler_params=pltpu.CompilerParams(
        use_tc_tiling_on_sc=False, needs_layout_passes=False
    ),
    scratch_types=[
        pltpu.VMEM((VOCAB,), jnp.float32),
        pltpu.VMEM((XRING, BATCH), jnp.int32),
        pltpu.VMEM((2, 32, 128), jnp.float32),
        pltpu.SemaphoreType.DMA((XRING,)),
        pltpu.SemaphoreType.DMA((2,)),
    ],
)
def _emb_gather(xT_hbm, tableT_hbm, outQ_hbm, erow_v, xr_v, ob_v,
                xsem, osem):
    cid = lax.axis_index("c")
    sid = lax.axis_index("s")
    wid = sid * NC + cid

    def xwait(slot):
        pltpu.make_async_copy(
            xT_hbm.at[0], xr_v.at[slot], xsem.at[slot]
        ).wait()

    def owait(slot):
        pltpu.make_async_copy(
            outQ_hbm.at[0, 0, :, 0], ob_v.at[slot], osem.at[slot]
        ).wait()

    def do_field(f, e, eb, k, xslot, oslot, first_store):
        xwait(xslot)
        if first_store is None:
            owait(oslot)
        else:
            @pl.when(first_store)
            def _():
                owait(oslot)
        U = 16
        for g0 in range(0, GROUPS, U):
            idxs = [
                xr_v[xslot, pl.ds((g0 + u) * 16, 16)] for u in range(U)
            ]
            vals = [plsc.load_gather(erow_v, [idxs[u]]) for u in range(U)]
            for u in range(U):
                g = g0 + u
                ob_v[oslot, g // 8, pl.ds((g % 8) * 16, 16)] = vals[u]
        pltpu.async_copy(
            ob_v.at[oslot], outQ_hbm.at[f, eb, :, k], osem.at[oslot]
        )

        @pl.when(f < FIELDS - XRING)
        def _():
            pltpu.async_copy(
                xT_hbm.at[f + XRING], xr_v.at[xslot], xsem.at[xslot]
            )

    def ebody(es, carry):
        e = wid * E_PER_W + es
        eb = e // 8
        k = e % 8
        pltpu.sync_copy(tableT_hbm.at[e], erow_v)
        for slot in range(XRING):
            pltpu.async_copy(xT_hbm.at[slot], xr_v.at[slot], xsem.at[slot])

        def fbody(f4, carry2):
            for sub in range(XRING):
                f = f4 * XRING + sub
                oslot = sub % 2
                pred = (
                    jnp.logical_or(es > 0, f4 >= 1) if sub < 2 else None
                )
                do_field(f, e, eb, k, sub, oslot, pred)
            return carry2

        lax.fori_loop(0, FIELDS // XRING, fbody, 0)
        for sub in range(FIELDS % XRING):
            f_tail = FIELDS - (FIELDS % XRING) + sub
            do_field(f_tail, e, eb, k, sub, sub % 2, None)
        return carry

    lax.fori_loop(0, E_PER_W, ebody, 0)
    owait(0)
    owait(1)


def kernel(x, table):
    outQ = _emb_gather(x.T, table.T)
    return outQ.transpose(2, 4, 0, 1, 3).reshape(BATCH, FIELDS, EMB)


# speedup vs baseline: 1.1330x; 1.1330x over previous
import functools

import jax
import jax.numpy as jnp
from jax import lax
from jax.experimental import pallas as pl
from jax.experimental.pallas import tpu as pltpu
from jax.experimental.pallas import tpu_sc as plsc

VOCAB = 100000
EMB = 64
BATCH = 4096
FIELDS = 26

NC = 2
NS = 16
NW = NC * NS
E_PER_W = EMB // NW
GROUPS = BATCH // 16
XRING = 4

_mesh = plsc.VectorSubcoreMesh(core_axis_name="c", subcore_axis_name="s")


@functools.partial(
    pl.kernel,
    mesh=_mesh,
    out_type=jax.ShapeDtypeStruct((FIELDS, 8, 32, 8, 128), jnp.float32),
    compiler_params=pltpu.CompilerParams(
        use_tc_tiling_on_sc=False, needs_layout_passes=False
    ),
    scratch_types=[
        pltpu.VMEM((VOCAB,), jnp.float32),
        pltpu.VMEM((XRING, BATCH), jnp.int32),
        pltpu.VMEM((2, 32, 128), jnp.float32),
        pltpu.SemaphoreType.DMA((XRING,)),
        pltpu.SemaphoreType.DMA((2,)),
    ],
)
def _emb_gather(xT_hbm, tableT_hbm, outQ_hbm, erow_v, xr_v, ob_v,
                xsem, osem):
    cid = lax.axis_index("c")
    sid = lax.axis_index("s")
    wid = sid * NC + cid

    def xwait(slot):
        pltpu.make_async_copy(
            xT_hbm.at[0], xr_v.at[slot], xsem.at[slot]
        ).wait()

    def owait(slot):
        pltpu.make_async_copy(
            outQ_hbm.at[0, 0, :, 0], ob_v.at[slot], osem.at[slot]
        ).wait()

    def do_field(f, e, eb, k, xslot, oslot, first_store):
        xwait(xslot)
        if first_store is None:
            owait(oslot)
        else:
            @pl.when(first_store)
            def _():
                owait(oslot)
        @plsc.parallel_loop(0, GROUPS, unroll=8)
        def _gather(g):
            idx = xr_v[xslot, pl.ds(g * 16, 16)]
            ob_v[oslot, g // 8, pl.ds((g % 8) * 16, 16)] = plsc.load_gather(
                erow_v, [idx]
            )
        pltpu.async_copy(
            ob_v.at[oslot], outQ_hbm.at[f, eb, :, k], osem.at[oslot]
        )

        @pl.when(f < FIELDS - XRING)
        def _():
            pltpu.async_copy(
                xT_hbm.at[f + XRING], xr_v.at[xslot], xsem.at[xslot]
            )

    def ebody(es, carry):
        e = wid * E_PER_W + es
        eb = e // 8
        k = e % 8
        pltpu.sync_copy(tableT_hbm.at[e], erow_v)
        for slot in range(XRING):
            pltpu.async_copy(xT_hbm.at[slot], xr_v.at[slot], xsem.at[slot])

        def fbody(f4, carry2):
            for sub in range(XRING):
                f = f4 * XRING + sub
                oslot = sub % 2
                pred = (
                    jnp.logical_or(es > 0, f4 >= 1) if sub < 2 else None
                )
                do_field(f, e, eb, k, sub, oslot, pred)
            return carry2

        lax.fori_loop(0, FIELDS // XRING, fbody, 0)
        for sub in range(FIELDS % XRING):
            f_tail = FIELDS - (FIELDS % XRING) + sub
            do_field(f_tail, e, eb, k, sub, sub % 2, None)
        return carry

    lax.fori_loop(0, E_PER_W, ebody, 0)
    owait(0)
    owait(1)


def kernel(x, table):
    outQ = _emb_gather(x.T, table.T)
    return outQ.transpose(2, 4, 0, 1, 3).reshape(BATCH, FIELDS, EMB)
